# baseline (speedup 1.0000x reference)
import functools

import jax
import jax.numpy as jnp
from jax import lax
from jax.experimental import pallas as pl
from jax.experimental.pallas import tpu as pltpu
from jax.experimental.pallas import tpu_sc as plsc

_BLOCK = 8192
_L = 16


def _mlp_kernel(comb_ref, w1_ref, b1_ref, w2_ref, b2_ref, out_ref):
    h = (jnp.dot(comb_ref[...], w1_ref[...], preferred_element_type=jnp.float32)
         + b1_ref[...][None, :])
    h = jnp.maximum(h, 0.0)
    out_ref[...] = (jnp.dot(h, w2_ref[...], preferred_element_type=jnp.float32)
                    + b2_ref[...][None, :])


def _make_sc_gather(B, vp, dp, vi, di, vc, dc):
    info = plsc.get_sparse_core_info()
    nc, ns = info.num_cores, info.num_subcores
    nw = nc * ns
    bpw = B // nw
    D = dp + di + dc
    mesh = plsc.VectorSubcoreMesh(core_axis_name="c", subcore_axis_name="s")

    @functools.partial(
        pl.kernel, mesh=mesh,
        compiler_params=pltpu.CompilerParams(use_tc_tiling_on_sc=False,
                                             needs_layout_passes=False),
        out_type=jax.ShapeDtypeStruct((B, D), jnp.float32),
        scratch_types=[
            pltpu.VMEM((bpw,), jnp.int32),
            pltpu.VMEM((bpw,), jnp.int32),
            pltpu.VMEM((bpw,), jnp.int32),
            pltpu.VMEM((vp, dp), jnp.float32),
            pltpu.VMEM((vi, di), jnp.float32),
            pltpu.VMEM((vc, dc), jnp.float32),
            pltpu.VMEM((bpw, D), jnp.float32),
            pltpu.SemaphoreType.DMA,
        ],
    )
    def sc_gather(pid_hbm, iid_hbm, cid_hbm, tp_hbm, ti_hbm, tc_hbm, out_hbm,
                  ip_v, ii_v, ic_v, tp_v, ti_v, tc_v, comb_v, sem):
        wid = lax.axis_index("s") * nc + lax.axis_index("c")
        base = wid * bpw
        pltpu.sync_copy(pid_hbm.at[pl.ds(base, bpw)], ip_v)
        pltpu.sync_copy(iid_hbm.at[pl.ds(base, bpw)], ii_v)
        pltpu.sync_copy(cid_hbm.at[pl.ds(base, bpw)], ic_v)
        pltpu.sync_copy(tp_hbm, tp_v)
        pltpu.sync_copy(ti_hbm, ti_v)
        pltpu.sync_copy(tc_hbm, tc_v)

        lane = lax.iota(jnp.int32, _L)

        def chunk(k, carry):
            e0 = k * _L
            e16 = lane + e0
            idx_p = ip_v[pl.ds(e0, _L)]
            idx_i = ii_v[pl.ds(e0, _L)]
            idx_c = ic_v[pl.ds(e0, _L)]
            for c in range(dp):
                col = jnp.full((_L,), c, jnp.int32)
                val = plsc.load_gather(tp_v, [idx_p, col])
                plsc.store_scatter(comb_v, [e16, col], val)
            for c in range(di):
                col = jnp.full((_L,), c, jnp.int32)
                val = plsc.load_gather(ti_v, [idx_i, col])
                plsc.store_scatter(comb_v, [e16, col + dp], val)
            for c in range(dc):
                col = jnp.full((_L,), c, jnp.int32)
                val = plsc.load_gather(tc_v, [idx_c, col])
                plsc.store_scatter(comb_v, [e16, col + (dp + di)], val)
            return carry

        lax.fori_loop(0, bpw // _L, chunk, 0)
        pltpu.sync_copy(comb_v, out_hbm.at[pl.ds(base, bpw)])

    return sc_gather


def kernel(platform_id, industry_id, cta_id, platform_table, industry_table,
           cta_table, W1, b1, W2, b2):
    B = platform_id.shape[0]
    vp, dp = platform_table.shape
    vi, di = industry_table.shape
    vc, dc = cta_table.shape
    D = dp + di + dc
    pid = platform_id.astype(jnp.int32)
    iid = industry_id.astype(jnp.int32)
    cid = cta_id.astype(jnp.int32)
    combined = _make_sc_gather(B, vp, dp, vi, di, vc, dc)(
        pid, iid, cid, platform_table, industry_table, cta_table)
    blk = min(_BLOCK, B)
    grid = B // blk
    d_out = W2.shape[1]
    return pl.pallas_call(
        _mlp_kernel,
        grid=(grid,),
        in_specs=[
            pl.BlockSpec((blk, D), lambda i: (i, 0)),
            pl.BlockSpec(W1.shape, lambda i: (0, 0)),
            pl.BlockSpec(b1.shape, lambda i: (0,)),
            pl.BlockSpec(W2.shape, lambda i: (0, 0)),
            pl.BlockSpec(b2.shape, lambda i: (0,)),
        ],
        out_specs=pl.BlockSpec((blk, d_out), lambda i: (i, 0)),
        out_shape=jax.ShapeDtypeStruct((B, d_out), jnp.float32),
    )(combined, W1, b1, W2, b2)


# speedup vs baseline: 3.9161x; 3.9161x over previous
import jax
import jax.numpy as jnp
from jax.experimental import pallas as pl

_BLOCK = 8192


def _fused_kernel_t(pid_ref, iid_ref, cid_ref, tp_ref, ti_ref, tc_ref,
                    w1_ref, b1_ref, w2_ref, b2_ref, out_ref):
    blk = pid_ref.shape[0]
    vp, dp = tp_ref.shape
    vi, di = ti_ref.shape
    vc, dc = tc_ref.shape
    rp = 8 * ((vp + 7) // 8)
    ri = 8 * ((vi + 7) // 8)
    rc = 8 * ((vc + 7) // 8)
    mp = jnp.dot(tp_ref[...], w1_ref[0:dp, :], preferred_element_type=jnp.float32)
    mi = jnp.dot(ti_ref[...], w1_ref[dp:dp + di, :], preferred_element_type=jnp.float32)
    mc = jnp.dot(tc_ref[...], w1_ref[dp + di:dp + di + dc, :], preferred_element_type=jnp.float32)
    pid = pid_ref[...][None, :]
    iid = iid_ref[...][None, :]
    cid = cid_ref[...][None, :]
    ohp = (jax.lax.broadcasted_iota(jnp.int32, (rp, blk), 0) == pid).astype(jnp.float32)
    ohi = (jax.lax.broadcasted_iota(jnp.int32, (ri, blk), 0) == iid).astype(jnp.float32)
    ohc = (jax.lax.broadcasted_iota(jnp.int32, (rc, blk), 0) == cid).astype(jnp.float32)
    hT = (jnp.dot(mp.T, ohp[0:vp, :], preferred_element_type=jnp.float32)
          + jnp.dot(mi.T, ohi[0:vi, :], preferred_element_type=jnp.float32)
          + jnp.dot(mc.T, ohc[0:vc, :], preferred_element_type=jnp.float32)
          + b1_ref[...][:, None])
    hT = jnp.maximum(hT, 0.0)
    outT = jnp.dot(w2_ref[...].T, hT, preferred_element_type=jnp.float32)
    out_ref[...] = outT.T + b2_ref[...][None, :]


def kernel(platform_id, industry_id, cta_id, platform_table, industry_table,
           cta_table, W1, b1, W2, b2):
    B = platform_id.shape[0]
    blk = min(_BLOCK, B)
    grid = B // blk
    pid2 = platform_id.astype(jnp.int32)
    iid2 = industry_id.astype(jnp.int32)
    cid2 = cta_id.astype(jnp.int32)
    d_out = W2.shape[1]
    return pl.pallas_call(
        _fused_kernel_t,
        grid=(grid,),
        in_specs=[
            pl.BlockSpec((blk,), lambda i: (i,)),
            pl.BlockSpec((blk,), lambda i: (i,)),
            pl.BlockSpec((blk,), lambda i: (i,)),
            pl.BlockSpec(platform_table.shape, lambda i: (0, 0)),
            pl.BlockSpec(industry_table.shape, lambda i: (0, 0)),
            pl.BlockSpec(cta_table.shape, lambda i: (0, 0)),
            pl.BlockSpec(W1.shape, lambda i: (0, 0)),
            pl.BlockSpec(b1.shape, lambda i: (0,)),
            pl.BlockSpec(W2.shape, lambda i: (0, 0)),
            pl.BlockSpec(b2.shape, lambda i: (0,)),
        ],
        out_specs=pl.BlockSpec((blk, d_out), lambda i: (i, 0)),
        out_shape=jax.ShapeDtypeStruct((B, d_out), jnp.float32),
    )(pid2, iid2, cid2, platform_table, industry_table, cta_table, W1, b1, W2, b2)


# speedup vs baseline: 4.6029x; 1.1754x over previous
import jax
import jax.numpy as jnp
from jax.experimental import pallas as pl
from jax.experimental.pallas import tpu as pltpu

_BLOCK = 8192


def _fused_kernel_t(pid_ref, iid_ref, cid_ref, tp_ref, ti_ref, tc_ref,
                    w1_ref, b1_ref, w2_ref, b2_ref, out_ref):
    blk = pid_ref.shape[0]
    vp, dp = tp_ref.shape
    vi, di = ti_ref.shape
    vc, dc = tc_ref.shape
    rp = 8 * ((vp + 7) // 8)
    ri = 8 * ((vi + 7) // 8)
    rc = 8 * ((vc + 7) // 8)
    mp = jnp.dot(tp_ref[...], w1_ref[0:dp, :], preferred_element_type=jnp.float32)
    mi = jnp.dot(ti_ref[...], w1_ref[dp:dp + di, :], preferred_element_type=jnp.float32)
    mc = jnp.dot(tc_ref[...], w1_ref[dp + di:dp + di + dc, :], preferred_element_type=jnp.float32)
    pid = pid_ref[...][None, :]
    iid = iid_ref[...][None, :]
    cid = cid_ref[...][None, :]
    ohp = (jax.lax.broadcasted_iota(jnp.int32, (rp, blk), 0) == pid).astype(jnp.float32)
    ohi = (jax.lax.broadcasted_iota(jnp.int32, (ri, blk), 0) == iid).astype(jnp.float32)
    ohc = (jax.lax.broadcasted_iota(jnp.int32, (rc, blk), 0) == cid).astype(jnp.float32)
    oh_all = jnp.concatenate([ohp, ohi, ohc], axis=0)
    d1 = w1_ref.shape[1]
    mt = jnp.concatenate([
        mp.T, jnp.zeros((d1, rp - vp), jnp.float32),
        mi.T, jnp.zeros((d1, ri - vi), jnp.float32),
        mc.T, jnp.zeros((d1, rc - vc), jnp.float32)], axis=1)
    hT = (jnp.dot(mt, oh_all, preferred_element_type=jnp.float32)
          + b1_ref[...][:, None])
    hT = jnp.maximum(hT, 0.0)
    out_ref[...] = (jnp.dot(hT.T, w2_ref[...], preferred_element_type=jnp.float32)
                    + b2_ref[...][None, :])


def kernel(platform_id, industry_id, cta_id, platform_table, industry_table,
           cta_table, W1, b1, W2, b2):
    B = platform_id.shape[0]
    blk = min(_BLOCK, B)
    grid = B // blk
    pid2 = platform_id.astype(jnp.int32)
    iid2 = industry_id.astype(jnp.int32)
    cid2 = cta_id.astype(jnp.int32)
    d_out = W2.shape[1]
    return pl.pallas_call(
        _fused_kernel_t,
        grid=(grid,),
        compiler_params=pltpu.CompilerParams(fuse_transposed_lhs_in_matmul=True),
        in_specs=[
            pl.BlockSpec((blk,), lambda i: (i,)),
            pl.BlockSpec((blk,), lambda i: (i,)),
            pl.BlockSpec((blk,), lambda i: (i,)),
            pl.BlockSpec(platform_table.shape, lambda i: (0, 0)),
            pl.BlockSpec(industry_table.shape, lambda i: (0, 0)),
            pl.BlockSpec(cta_table.shape, lambda i: (0, 0)),
            pl.BlockSpec(W1.shape, lambda i: (0, 0)),
            pl.BlockSpec(b1.shape, lambda i: (0,)),
            pl.BlockSpec(W2.shape, lambda i: (0, 0)),
            pl.BlockSpec(b2.shape, lambda i: (0,)),
        ],
        out_specs=pl.BlockSpec((blk, d_out), lambda i: (i, 0)),
        out_shape=jax.ShapeDtypeStruct((B, d_out), jnp.float32),
    )(pid2, iid2, cid2, platform_table, industry_table, cta_table, W1, b1, W2, b2)
